# baseline (speedup 1.0000x reference)
import functools

import jax
import jax.numpy as jnp
from jax import lax
from jax.experimental import pallas as pl
from jax.experimental.pallas import tpu as pltpu
from jax.experimental.pallas import tpu_sc as plsc

NC = 2
NS = 16
NW = NC * NS
BSZ = 128
HW = 16


def _mesh():
    return plsc.VectorSubcoreMesh(core_axis_name="c", subcore_axis_name="s")


def _make_hist(n_pad, nb):
    rows = n_pad // NS

    @functools.partial(
        pl.kernel,
        out_type=jax.ShapeDtypeStruct((NC, n_pad, HW), jnp.float32),
        mesh=_mesh(),
        compiler_params=pltpu.CompilerParams(use_tc_tiling_on_sc=False),
        scratch_types=[
            pltpu.VMEM((nb, BSZ), jnp.int32),
            pltpu.VMEM((BSZ, HW), jnp.float32),
            pltpu.VMEM_SHARED((n_pad, HW), jnp.float32),
        ],
    )
    def hist(dst_hbm, ones_hbm, z_hbm, out_hbm, dst_v, ones_v, acc_s):
        c = lax.axis_index("c")
        s = lax.axis_index("s")
        wid = s * NC + c
        pltpu.sync_copy(dst_hbm.at[wid], dst_v)
        pltpu.sync_copy(ones_hbm, ones_v)
        base = s * rows
        pltpu.sync_copy(z_hbm.at[pl.ds(base, rows)], acc_s.at[pl.ds(base, rows)])
        plsc.subcore_barrier()

        def body(j, carry):
            pltpu.sync_copy(ones_v, acc_s.at[dst_v.at[j]], add=True)
            return carry

        lax.fori_loop(0, nb, body, 0)
        plsc.subcore_barrier()
        pltpu.sync_copy(acc_s.at[pl.ds(base, rows)], out_hbm.at[c, pl.ds(base, rows)])

    return hist


K = 2


def _make_prop(n_pad, d, nb):
    rows = n_pad // NS
    assert nb % 2 == 0

    @functools.partial(
        pl.kernel,
        out_type=jax.ShapeDtypeStruct((NC, n_pad, d), jnp.float32),
        mesh=_mesh(),
        compiler_params=pltpu.CompilerParams(use_tc_tiling_on_sc=False),
        scratch_types=[
            pltpu.VMEM((nb, BSZ), jnp.int32),
            pltpu.VMEM((nb, BSZ), jnp.int32),
            [pltpu.VMEM((BSZ, d), jnp.float32) for _ in range(K)],
            pltpu.VMEM_SHARED((n_pad, d), jnp.float32),
            [pltpu.SemaphoreType.DMA for _ in range(K)],
        ],
    )
    def prop(h_hbm, src_hbm, dst_hbm, z_hbm, out_hbm, src_v, dst_v, rows_v,
             acc_s, gsems):
        c = lax.axis_index("c")
        s = lax.axis_index("s")
        wid = s * NC + c
        pltpu.sync_copy(src_hbm.at[wid], src_v)
        pltpu.sync_copy(dst_hbm.at[wid], dst_v)
        base = s * rows
        pltpu.sync_copy(z_hbm.at[pl.ds(base, rows)], acc_s.at[pl.ds(base, rows)])
        plsc.subcore_barrier()

        def body(j, carry):
            pltpu.async_copy(h_hbm.at[src_v.at[j]], rows_v[0], gsems[0]).wait()
            pltpu.sync_copy(rows_v[0], acc_s.at[dst_v.at[j]], add=True)
            return carry

        lax.fori_loop(0, nb, body, 0)
        plsc.subcore_barrier()
        pltpu.sync_copy(acc_s.at[pl.ds(base, rows)], out_hbm.at[c, pl.ds(base, rows)])

    return prop


def _tc_matmul(x, w):
    def body(x_ref, w_ref, o_ref):
        o_ref[...] = jnp.dot(x_ref[...], w_ref[...],
                             preferred_element_type=jnp.float32)

    return pl.pallas_call(
        body,
        out_shape=jax.ShapeDtypeStruct((x.shape[0], w.shape[1]), jnp.float32),
    )(x, w)


def _tc_scale1(hp, h):
    def body(hp_ref, h_ref, dinv_ref, h1p_ref):
        deg = hp_ref[0, :, 0] + hp_ref[1, :, 0] + 1.0
        dinv = lax.rsqrt(deg)
        dinv_ref[...] = dinv[:, None]
        h1p_ref[...] = h_ref[...] * dinv[:, None]

    n = h.shape[0]
    return pl.pallas_call(
        body,
        out_shape=(
            jax.ShapeDtypeStruct((n, 1), jnp.float32),
            jax.ShapeDtypeStruct(h.shape, jnp.float32),
        ),
    )(hp, h)


def _tc_stage2(p, hprev, dinv, b, w):
    def body(p_ref, hp_ref, dinv_ref, b_ref, w_ref, o_ref):
        dv = dinv_ref[...]
        tot = (p_ref[0] + p_ref[1] + hp_ref[...]) * dv + b_ref[...]
        hact = jnp.maximum(tot, 0.0)
        o_ref[...] = jnp.dot(hact, w_ref[...],
                             preferred_element_type=jnp.float32) * dv

    n = hprev.shape[0]
    return pl.pallas_call(
        body,
        out_shape=jax.ShapeDtypeStruct((n, w.shape[1]), jnp.float32),
    )(p, hprev, dinv, b.reshape(1, -1), w)


def _tc_stage3(p, hprev, dinv, b, w, bo):
    def body(p_ref, hp_ref, dinv_ref, b_ref, w_ref, bo_ref, o_ref):
        dv = dinv_ref[...]
        tot = (p_ref[0] + p_ref[1] + hp_ref[...]) * dv + b_ref[...]
        hact = jnp.maximum(tot, 0.0)
        o_ref[...] = jnp.dot(hact, w_ref[...],
                             preferred_element_type=jnp.float32) + bo_ref[...]

    n = hprev.shape[0]
    return pl.pallas_call(
        body,
        out_shape=jax.ShapeDtypeStruct((n, w.shape[1]), jnp.float32),
    )(p, hprev, dinv, b.reshape(1, -1), w, bo.reshape(1, -1))


def kernel(x, edge_index, W1, b1, W2, b2, W_out, b_out):
    n, d_in = x.shape
    e = edge_index.shape[1]
    n_pad = ((n + NS * 8 - 1) // (NS * 8)) * (NS * 8)
    nb = (e + NW * BSZ - 1) // (NW * BSZ)
    nb = ((nb + K - 1) // K) * K
    e_pad = NW * BSZ * nb

    xp = jnp.pad(x, ((0, n_pad - n), (0, 0)))
    src = jnp.concatenate(
        [edge_index[0], jnp.zeros((e_pad - e,), jnp.int32)]).reshape(NW, nb, BSZ)
    dst = jnp.concatenate(
        [edge_index[1], jnp.full((e_pad - e,), n, jnp.int32)]).reshape(NW, nb, BSZ)

    ones_hw = jnp.ones((BSZ, HW), jnp.float32)
    z_hw = jnp.zeros((n_pad, HW), jnp.float32)
    z1 = jnp.zeros((n_pad, W1.shape[1]), jnp.float32)
    z2 = jnp.zeros((n_pad, W2.shape[1]), jnp.float32)

    hp = _make_hist(n_pad, nb)(dst, ones_hw, z_hw)
    h = _tc_matmul(xp, W1)
    dinv, h1p = _tc_scale1(hp, h)
    p1 = _make_prop(n_pad, W1.shape[1], nb)(h1p, src, dst, z1)
    h2p = _tc_stage2(p1, h1p, dinv, b1, W2)
    p2 = _make_prop(n_pad, W2.shape[1], nb)(h2p, src, dst, z2)
    out = _tc_stage3(p2, h2p, dinv, b2, W_out, b_out)
    return out[:n]


# speedup vs baseline: 1.0010x; 1.0010x over previous
import functools

import jax
import jax.numpy as jnp
from jax import lax
from jax.experimental import pallas as pl
from jax.experimental.pallas import tpu as pltpu
from jax.experimental.pallas import tpu_sc as plsc

NC = 2
NS = 16
NW = NC * NS
BSZ = 128
HW = 16


def _mesh():
    return plsc.VectorSubcoreMesh(core_axis_name="c", subcore_axis_name="s")


def _make_hist(n_pad, nb):
    rows = n_pad // NS

    @functools.partial(
        pl.kernel,
        out_type=jax.ShapeDtypeStruct((NC, n_pad, HW), jnp.float32),
        mesh=_mesh(),
        compiler_params=pltpu.CompilerParams(use_tc_tiling_on_sc=False),
        scratch_types=[
            pltpu.VMEM((nb, BSZ), jnp.int32),
            pltpu.VMEM((BSZ, HW), jnp.float32),
            pltpu.VMEM_SHARED((n_pad, HW), jnp.float32),
        ],
    )
    def hist(dst_hbm, ones_hbm, z_hbm, out_hbm, dst_v, ones_v, acc_s):
        c = lax.axis_index("c")
        s = lax.axis_index("s")
        wid = s * NC + c
        pltpu.sync_copy(dst_hbm.at[wid], dst_v)
        pltpu.sync_copy(ones_hbm, ones_v)
        base = s * rows
        pltpu.sync_copy(z_hbm.at[pl.ds(base, rows)], acc_s.at[pl.ds(base, rows)])
        plsc.subcore_barrier()

        def body(j, carry):
            pltpu.sync_copy(ones_v, acc_s.at[dst_v.at[j]], add=True)
            return carry

        lax.fori_loop(0, nb, body, 0)
        plsc.subcore_barrier()
        pltpu.sync_copy(acc_s.at[pl.ds(base, rows)], out_hbm.at[c, pl.ds(base, rows)])

    return hist


K = 2


def _make_prop(n_pad, d, nb):
    rows = n_pad // NS
    assert nb % 2 == 0

    @functools.partial(
        pl.kernel,
        out_type=jax.ShapeDtypeStruct((NC, n_pad, d), jnp.float32),
        mesh=_mesh(),
        compiler_params=pltpu.CompilerParams(use_tc_tiling_on_sc=False),
        scratch_types=[
            pltpu.VMEM((nb, BSZ), jnp.int32),
            pltpu.VMEM((nb, BSZ), jnp.int32),
            pltpu.VMEM((BSZ, d), jnp.float32),
            pltpu.VMEM_SHARED((n_pad, d), jnp.float32),
        ],
    )
    def prop(h_hbm, src_hbm, dst_hbm, z_hbm, out_hbm, src_v, dst_v, rows_v,
             acc_s):
        c = lax.axis_index("c")
        s = lax.axis_index("s")
        wid = s * NC + c
        pltpu.sync_copy(src_hbm.at[wid], src_v)
        pltpu.sync_copy(dst_hbm.at[wid], dst_v)
        base = s * rows
        pltpu.sync_copy(z_hbm.at[pl.ds(base, rows)], acc_s.at[pl.ds(base, rows)])
        plsc.subcore_barrier()

        def body(j, carry):
            pltpu.sync_copy(h_hbm.at[src_v.at[j]], rows_v)
            pltpu.sync_copy(rows_v, acc_s.at[dst_v.at[j]], add=True)
            return carry

        lax.fori_loop(0, nb, body, 0)
        plsc.subcore_barrier()
        pltpu.sync_copy(acc_s.at[pl.ds(base, rows)], out_hbm.at[c, pl.ds(base, rows)])

    return prop


def _tc_matmul(x, w):
    def body(x_ref, w_ref, o_ref):
        o_ref[...] = jnp.dot(x_ref[...], w_ref[...],
                             preferred_element_type=jnp.float32)

    return pl.pallas_call(
        body,
        out_shape=jax.ShapeDtypeStruct((x.shape[0], w.shape[1]), jnp.float32),
    )(x, w)


def _tc_scale1(hp, h):
    def body(hp_ref, h_ref, dinv_ref, h1p_ref):
        deg = hp_ref[0, :, 0] + hp_ref[1, :, 0] + 1.0
        dinv = lax.rsqrt(deg)
        dinv_ref[...] = dinv[:, None]
        h1p_ref[...] = h_ref[...] * dinv[:, None]

    n = h.shape[0]
    return pl.pallas_call(
        body,
        out_shape=(
            jax.ShapeDtypeStruct((n, 1), jnp.float32),
            jax.ShapeDtypeStruct(h.shape, jnp.float32),
        ),
    )(hp, h)


def _tc_stage2(p, hprev, dinv, b, w):
    def body(p_ref, hp_ref, dinv_ref, b_ref, w_ref, o_ref):
        dv = dinv_ref[...]
        tot = (p_ref[0] + p_ref[1] + hp_ref[...]) * dv + b_ref[...]
        hact = jnp.maximum(tot, 0.0)
        o_ref[...] = jnp.dot(hact, w_ref[...],
                             preferred_element_type=jnp.float32) * dv

    n = hprev.shape[0]
    return pl.pallas_call(
        body,
        out_shape=jax.ShapeDtypeStruct((n, w.shape[1]), jnp.float32),
    )(p, hprev, dinv, b.reshape(1, -1), w)


def _tc_stage3(p, hprev, dinv, b, w, bo):
    def body(p_ref, hp_ref, dinv_ref, b_ref, w_ref, bo_ref, o_ref):
        dv = dinv_ref[...]
        tot = (p_ref[0] + p_ref[1] + hp_ref[...]) * dv + b_ref[...]
        hact = jnp.maximum(tot, 0.0)
        o_ref[...] = jnp.dot(hact, w_ref[...],
                             preferred_element_type=jnp.float32) + bo_ref[...]

    n = hprev.shape[0]
    return pl.pallas_call(
        body,
        out_shape=jax.ShapeDtypeStruct((n, w.shape[1]), jnp.float32),
    )(p, hprev, dinv, b.reshape(1, -1), w, bo.reshape(1, -1))


def kernel(x, edge_index, W1, b1, W2, b2, W_out, b_out):
    n, d_in = x.shape
    e = edge_index.shape[1]
    n_pad = ((n + NS * 8 - 1) // (NS * 8)) * (NS * 8)
    nb = (e + NW * BSZ - 1) // (NW * BSZ)
    nb = ((nb + K - 1) // K) * K
    e_pad = NW * BSZ * nb

    xp = jnp.pad(x, ((0, n_pad - n), (0, 0)))
    src = jnp.concatenate(
        [edge_index[0], jnp.zeros((e_pad - e,), jnp.int32)]).reshape(NW, nb, BSZ)
    dst = jnp.concatenate(
        [edge_index[1], jnp.full((e_pad - e,), n, jnp.int32)]).reshape(NW, nb, BSZ)

    ones_hw = jnp.ones((BSZ, HW), jnp.float32)
    z_hw = jnp.zeros((n_pad, HW), jnp.float32)
    z1 = jnp.zeros((n_pad, W1.shape[1]), jnp.float32)
    z2 = jnp.zeros((n_pad, W2.shape[1]), jnp.float32)

    hp = _make_hist(n_pad, nb)(dst, ones_hw, z_hw)
    h = _tc_matmul(xp, W1)
    dinv, h1p = _tc_scale1(hp, h)
    p1 = _make_prop(n_pad, W1.shape[1], nb)(h1p, src, dst, z1)
    h2p = _tc_stage2(p1, h1p, dinv, b1, W2)
    p2 = _make_prop(n_pad, W2.shape[1], nb)(h2p, src, dst, z2)
    out = _tc_stage3(p2, h2p, dinv, b2, W_out, b_out)
    return out[:n]


# speedup vs baseline: 1.9385x; 1.9366x over previous
import functools

import jax
import jax.numpy as jnp
from jax import lax
from jax.experimental import pallas as pl
from jax.experimental.pallas import tpu as pltpu
from jax.experimental.pallas import tpu_sc as plsc

NC = 2
NS = 16
NW = NC * NS
BSZ = 128
HW = 16


def _mesh():
    return plsc.VectorSubcoreMesh(core_axis_name="c", subcore_axis_name="s")


def _make_hist(n_pad, nb):
    rows = n_pad // NS

    @functools.partial(
        pl.kernel,
        out_type=jax.ShapeDtypeStruct((NC, n_pad, HW), jnp.float32),
        mesh=_mesh(),
        compiler_params=pltpu.CompilerParams(use_tc_tiling_on_sc=False),
        scratch_types=[
            pltpu.VMEM((nb, BSZ), jnp.int32),
            pltpu.VMEM((BSZ, HW), jnp.float32),
            pltpu.VMEM_SHARED((n_pad, HW), jnp.float32),
        ],
    )
    def hist(dst_hbm, ones_hbm, z_hbm, out_hbm, dst_v, ones_v, acc_s):
        c = lax.axis_index("c")
        s = lax.axis_index("s")
        wid = s * NC + c
        pltpu.sync_copy(dst_hbm.at[wid], dst_v)
        pltpu.sync_copy(ones_hbm, ones_v)
        base = s * rows
        pltpu.sync_copy(z_hbm.at[pl.ds(base, rows)], acc_s.at[pl.ds(base, rows)])
        plsc.subcore_barrier()

        def body(j, carry):
            pltpu.sync_copy(ones_v, acc_s.at[dst_v.at[j]], add=True)
            return carry

        lax.fori_loop(0, nb, body, 0)
        plsc.subcore_barrier()
        pltpu.sync_copy(acc_s.at[pl.ds(base, rows)], out_hbm.at[c, pl.ds(base, rows)])

    return hist


K = 2


def _make_prop(n_pad, d, nb):
    rows = n_pad // NS
    assert nb % 2 == 0

    @functools.partial(
        pl.kernel,
        out_type=jax.ShapeDtypeStruct((NC, n_pad, d), jnp.float32),
        mesh=_mesh(),
        compiler_params=pltpu.CompilerParams(use_tc_tiling_on_sc=False),
        scratch_types=[
            pltpu.VMEM((nb, BSZ), jnp.int32),
            pltpu.VMEM((nb, BSZ), jnp.int32),
            pltpu.VMEM((BSZ, d), jnp.float32),
            pltpu.VMEM_SHARED((n_pad, d), jnp.float32),
            pltpu.VMEM_SHARED((n_pad, d), jnp.float32),
        ],
    )
    def prop(h_hbm, src_hbm, dst_hbm, z_hbm, out_hbm, src_v, dst_v, rows_v,
             acc_s, tab_s):
        c = lax.axis_index("c")
        s = lax.axis_index("s")
        wid = s * NC + c
        pltpu.sync_copy(src_hbm.at[wid], src_v)
        pltpu.sync_copy(dst_hbm.at[wid], dst_v)
        base = s * rows
        pltpu.sync_copy(z_hbm.at[pl.ds(base, rows)], acc_s.at[pl.ds(base, rows)])
        pltpu.sync_copy(h_hbm.at[pl.ds(base, rows)], tab_s.at[pl.ds(base, rows)])
        plsc.subcore_barrier()

        def body(j, carry):
            pltpu.sync_copy(tab_s.at[src_v.at[j]], rows_v)
            pltpu.sync_copy(rows_v, acc_s.at[dst_v.at[j]], add=True)
            return carry

        lax.fori_loop(0, nb, body, 0)
        plsc.subcore_barrier()
        pltpu.sync_copy(acc_s.at[pl.ds(base, rows)], out_hbm.at[c, pl.ds(base, rows)])

    return prop


def _tc_matmul(x, w):
    def body(x_ref, w_ref, o_ref):
        o_ref[...] = jnp.dot(x_ref[...], w_ref[...],
                             preferred_element_type=jnp.float32)

    return pl.pallas_call(
        body,
        out_shape=jax.ShapeDtypeStruct((x.shape[0], w.shape[1]), jnp.float32),
    )(x, w)


def _tc_scale1(hp, h):
    def body(hp_ref, h_ref, dinv_ref, h1p_ref):
        deg = hp_ref[0, :, 0] + hp_ref[1, :, 0] + 1.0
        dinv = lax.rsqrt(deg)
        dinv_ref[...] = dinv[:, None]
        h1p_ref[...] = h_ref[...] * dinv[:, None]

    n = h.shape[0]
    return pl.pallas_call(
        body,
        out_shape=(
            jax.ShapeDtypeStruct((n, 1), jnp.float32),
            jax.ShapeDtypeStruct(h.shape, jnp.float32),
        ),
    )(hp, h)


def _tc_stage2(p, hprev, dinv, b, w):
    def body(p_ref, hp_ref, dinv_ref, b_ref, w_ref, o_ref):
        dv = dinv_ref[...]
        tot = (p_ref[0] + p_ref[1] + hp_ref[...]) * dv + b_ref[...]
        hact = jnp.maximum(tot, 0.0)
        o_ref[...] = jnp.dot(hact, w_ref[...],
                             preferred_element_type=jnp.float32) * dv

    n = hprev.shape[0]
    return pl.pallas_call(
        body,
        out_shape=jax.ShapeDtypeStruct((n, w.shape[1]), jnp.float32),
    )(p, hprev, dinv, b.reshape(1, -1), w)


def _tc_stage3(p, hprev, dinv, b, w, bo):
    def body(p_ref, hp_ref, dinv_ref, b_ref, w_ref, bo_ref, o_ref):
        dv = dinv_ref[...]
        tot = (p_ref[0] + p_ref[1] + hp_ref[...]) * dv + b_ref[...]
        hact = jnp.maximum(tot, 0.0)
        o_ref[...] = jnp.dot(hact, w_ref[...],
                             preferred_element_type=jnp.float32) + bo_ref[...]

    n = hprev.shape[0]
    return pl.pallas_call(
        body,
        out_shape=jax.ShapeDtypeStruct((n, w.shape[1]), jnp.float32),
    )(p, hprev, dinv, b.reshape(1, -1), w, bo.reshape(1, -1))


def kernel(x, edge_index, W1, b1, W2, b2, W_out, b_out):
    n, d_in = x.shape
    e = edge_index.shape[1]
    n_pad = ((n + NS * 8 - 1) // (NS * 8)) * (NS * 8)
    nb = (e + NW * BSZ - 1) // (NW * BSZ)
    nb = ((nb + K - 1) // K) * K
    e_pad = NW * BSZ * nb

    xp = jnp.pad(x, ((0, n_pad - n), (0, 0)))
    src = jnp.concatenate(
        [edge_index[0], jnp.zeros((e_pad - e,), jnp.int32)]).reshape(NW, nb, BSZ)
    dst = jnp.concatenate(
        [edge_index[1], jnp.full((e_pad - e,), n, jnp.int32)]).reshape(NW, nb, BSZ)

    ones_hw = jnp.ones((BSZ, HW), jnp.float32)
    z_hw = jnp.zeros((n_pad, HW), jnp.float32)
    z1 = jnp.zeros((n_pad, W1.shape[1]), jnp.float32)
    z2 = jnp.zeros((n_pad, W2.shape[1]), jnp.float32)

    hp = _make_hist(n_pad, nb)(dst, ones_hw, z_hw)
    h = _tc_matmul(xp, W1)
    dinv, h1p = _tc_scale1(hp, h)
    p1 = _make_prop(n_pad, W1.shape[1], nb)(h1p, src, dst, z1)
    h2p = _tc_stage2(p1, h1p, dinv, b1, W2)
    p2 = _make_prop(n_pad, W2.shape[1], nb)(h2p, src, dst, z2)
    out = _tc_stage3(p2, h2p, dinv, b2, W_out, b_out)
    return out[:n]


# speedup vs baseline: 2.0474x; 1.0562x over previous
import functools

import jax
import jax.numpy as jnp
from jax import lax
from jax.experimental import pallas as pl
from jax.experimental.pallas import tpu as pltpu
from jax.experimental.pallas import tpu_sc as plsc

NC = 2
NS = 16
NW = NC * NS
BSZ = 128
HW = 16


def _mesh():
    return plsc.VectorSubcoreMesh(core_axis_name="c", subcore_axis_name="s")


def _make_hist(n_pad, nb):
    rows = n_pad // NS

    @functools.partial(
        pl.kernel,
        out_type=jax.ShapeDtypeStruct((NC, n_pad, HW), jnp.float32),
        mesh=_mesh(),
        compiler_params=pltpu.CompilerParams(use_tc_tiling_on_sc=False),
        scratch_types=[
            pltpu.VMEM((nb, BSZ), jnp.int32),
            pltpu.VMEM((BSZ, HW), jnp.float32),
            pltpu.VMEM_SHARED((n_pad, HW), jnp.float32),
        ],
    )
    def hist(dst_hbm, ones_hbm, z_hbm, out_hbm, dst_v, ones_v, acc_s):
        c = lax.axis_index("c")
        s = lax.axis_index("s")
        wid = s * NC + c
        pltpu.sync_copy(dst_hbm.at[wid], dst_v)
        pltpu.sync_copy(ones_hbm, ones_v)
        base = s * rows
        pltpu.sync_copy(z_hbm.at[pl.ds(base, rows)], acc_s.at[pl.ds(base, rows)])
        plsc.subcore_barrier()

        def body(j, carry):
            pltpu.sync_copy(ones_v, acc_s.at[dst_v.at[j]], add=True)
            return carry

        lax.fori_loop(0, nb, body, 0)
        plsc.subcore_barrier()
        pltpu.sync_copy(acc_s.at[pl.ds(base, rows)], out_hbm.at[c, pl.ds(base, rows)])

    return hist


K = 2


def _make_prop(n_pad, d, nb):
    rows = n_pad // NS
    assert nb % 2 == 0

    @functools.partial(
        pl.kernel,
        out_type=jax.ShapeDtypeStruct((NC, n_pad, d), jnp.float32),
        mesh=_mesh(),
        compiler_params=pltpu.CompilerParams(use_tc_tiling_on_sc=False),
        scratch_types=[
            pltpu.VMEM((nb, BSZ), jnp.int32),
            pltpu.VMEM((nb, BSZ), jnp.int32),
            pltpu.VMEM((BSZ, d), jnp.float32),
            pltpu.VMEM((BSZ, d), jnp.float32),
            pltpu.VMEM_SHARED((n_pad, d), jnp.float32),
            pltpu.VMEM_SHARED((n_pad, d), jnp.float32),
            pltpu.SemaphoreType.DMA,
        ],
    )
    def prop(h_hbm, src_hbm, dst_hbm, z_hbm, out_hbm, src_v, dst_v, rows_a,
             rows_b, acc_s, tab_s, gsem):
        c = lax.axis_index("c")
        s = lax.axis_index("s")
        wid = s * NC + c
        pltpu.sync_copy(src_hbm.at[wid], src_v)
        pltpu.sync_copy(dst_hbm.at[wid], dst_v)
        base = s * rows
        pltpu.sync_copy(z_hbm.at[pl.ds(base, rows)], acc_s.at[pl.ds(base, rows)])
        pltpu.sync_copy(h_hbm.at[pl.ds(base, rows)], tab_s.at[pl.ds(base, rows)])
        plsc.subcore_barrier()

        def body(i, carry):
            j0 = 2 * i
            j1 = j0 + 1
            pltpu.sync_copy(tab_s.at[src_v.at[j0]], rows_a)
            h = pltpu.async_copy(tab_s.at[src_v.at[j1]], rows_b, gsem)
            pltpu.sync_copy(rows_a, acc_s.at[dst_v.at[j0]], add=True)
            h.wait()
            pltpu.sync_copy(rows_b, acc_s.at[dst_v.at[j1]], add=True)
            return carry

        lax.fori_loop(0, nb // 2, body, 0)
        plsc.subcore_barrier()
        pltpu.sync_copy(acc_s.at[pl.ds(base, rows)], out_hbm.at[c, pl.ds(base, rows)])

    return prop


def _tc_matmul(x, w):
    def body(x_ref, w_ref, o_ref):
        o_ref[...] = jnp.dot(x_ref[...], w_ref[...],
                             preferred_element_type=jnp.float32)

    return pl.pallas_call(
        body,
        out_shape=jax.ShapeDtypeStruct((x.shape[0], w.shape[1]), jnp.float32),
    )(x, w)


def _tc_scale1(hp, h):
    def body(hp_ref, h_ref, dinv_ref, h1p_ref):
        deg = hp_ref[0, :, 0] + hp_ref[1, :, 0] + 1.0
        dinv = lax.rsqrt(deg)
        dinv_ref[...] = dinv[:, None]
        h1p_ref[...] = h_ref[...] * dinv[:, None]

    n = h.shape[0]
    return pl.pallas_call(
        body,
        out_shape=(
            jax.ShapeDtypeStruct((n, 1), jnp.float32),
            jax.ShapeDtypeStruct(h.shape, jnp.float32),
        ),
    )(hp, h)


def _tc_stage2(p, hprev, dinv, b, w):
    def body(p_ref, hp_ref, dinv_ref, b_ref, w_ref, o_ref):
        dv = dinv_ref[...]
        tot = (p_ref[0] + p_ref[1] + hp_ref[...]) * dv + b_ref[...]
        hact = jnp.maximum(tot, 0.0)
        o_ref[...] = jnp.dot(hact, w_ref[...],
                             preferred_element_type=jnp.float32) * dv

    n = hprev.shape[0]
    return pl.pallas_call(
        body,
        out_shape=jax.ShapeDtypeStruct((n, w.shape[1]), jnp.float32),
    )(p, hprev, dinv, b.reshape(1, -1), w)


def _tc_stage3(p, hprev, dinv, b, w, bo):
    def body(p_ref, hp_ref, dinv_ref, b_ref, w_ref, bo_ref, o_ref):
        dv = dinv_ref[...]
        tot = (p_ref[0] + p_ref[1] + hp_ref[...]) * dv + b_ref[...]
        hact = jnp.maximum(tot, 0.0)
        o_ref[...] = jnp.dot(hact, w_ref[...],
                             preferred_element_type=jnp.float32) + bo_ref[...]

    n = hprev.shape[0]
    return pl.pallas_call(
        body,
        out_shape=jax.ShapeDtypeStruct((n, w.shape[1]), jnp.float32),
    )(p, hprev, dinv, b.reshape(1, -1), w, bo.reshape(1, -1))


def kernel(x, edge_index, W1, b1, W2, b2, W_out, b_out):
    n, d_in = x.shape
    e = edge_index.shape[1]
    n_pad = ((n + NS * 8 - 1) // (NS * 8)) * (NS * 8)
    nb = (e + NW * BSZ - 1) // (NW * BSZ)
    nb = ((nb + K - 1) // K) * K
    e_pad = NW * BSZ * nb

    xp = jnp.pad(x, ((0, n_pad - n), (0, 0)))
    src = jnp.concatenate(
        [edge_index[0], jnp.zeros((e_pad - e,), jnp.int32)]).reshape(NW, nb, BSZ)
    dst = jnp.concatenate(
        [edge_index[1], jnp.full((e_pad - e,), n, jnp.int32)]).reshape(NW, nb, BSZ)

    ones_hw = jnp.ones((BSZ, HW), jnp.float32)
    z_hw = jnp.zeros((n_pad, HW), jnp.float32)
    z1 = jnp.zeros((n_pad, W1.shape[1]), jnp.float32)
    z2 = jnp.zeros((n_pad, W2.shape[1]), jnp.float32)

    hp = _make_hist(n_pad, nb)(dst, ones_hw, z_hw)
    h = _tc_matmul(xp, W1)
    dinv, h1p = _tc_scale1(hp, h)
    p1 = _make_prop(n_pad, W1.shape[1], nb)(h1p, src, dst, z1)
    h2p = _tc_stage2(p1, h1p, dinv, b1, W2)
    p2 = _make_prop(n_pad, W2.shape[1], nb)(h2p, src, dst, z2)
    out = _tc_stage3(p2, h2p, dinv, b2, W_out, b_out)
    return out[:n]


# speedup vs baseline: 2.0490x; 1.0008x over previous
import functools

import jax
import jax.numpy as jnp
from jax import lax
from jax.experimental import pallas as pl
from jax.experimental.pallas import tpu as pltpu
from jax.experimental.pallas import tpu_sc as plsc

NC = 2
NS = 16
NW = NC * NS
BSZ = 128
HW = 16


def _mesh():
    return plsc.VectorSubcoreMesh(core_axis_name="c", subcore_axis_name="s")


def _make_hist(n_pad, nb):
    rows = n_pad // NS

    @functools.partial(
        pl.kernel,
        out_type=jax.ShapeDtypeStruct((NC, n_pad, HW), jnp.float32),
        mesh=_mesh(),
        compiler_params=pltpu.CompilerParams(use_tc_tiling_on_sc=False),
        scratch_types=[
            pltpu.VMEM((nb, BSZ), jnp.int32),
            pltpu.VMEM((BSZ, HW), jnp.float32),
            pltpu.VMEM_SHARED((n_pad, HW), jnp.float32),
        ],
    )
    def hist(dst_hbm, ones_hbm, z_hbm, out_hbm, dst_v, ones_v, acc_s):
        c = lax.axis_index("c")
        s = lax.axis_index("s")
        wid = s * NC + c
        pltpu.sync_copy(dst_hbm.at[wid], dst_v)
        pltpu.sync_copy(ones_hbm, ones_v)
        base = s * rows
        pltpu.sync_copy(z_hbm.at[pl.ds(base, rows)], acc_s.at[pl.ds(base, rows)])
        plsc.subcore_barrier()

        def body(j, carry):
            pltpu.sync_copy(ones_v, acc_s.at[dst_v.at[j]], add=True)
            return carry

        lax.fori_loop(0, nb, body, 0)
        plsc.subcore_barrier()
        pltpu.sync_copy(acc_s.at[pl.ds(base, rows)], out_hbm.at[c, pl.ds(base, rows)])

    return hist


K = 2


def _make_prop(n_pad, d, nb):
    rows = n_pad // NS
    assert nb % 2 == 0

    @functools.partial(
        pl.kernel,
        out_type=jax.ShapeDtypeStruct((NC, n_pad, d), jnp.float32),
        mesh=_mesh(),
        compiler_params=pltpu.CompilerParams(use_tc_tiling_on_sc=False),
        scratch_types=[
            pltpu.VMEM((nb, BSZ), jnp.int32),
            pltpu.VMEM((nb, BSZ), jnp.int32),
            pltpu.VMEM((BSZ, d), jnp.float32),
            pltpu.VMEM((BSZ, d), jnp.float32),
            pltpu.VMEM_SHARED((n_pad, d), jnp.float32),
            pltpu.VMEM_SHARED((n_pad, d), jnp.float32),
            pltpu.SemaphoreType.DMA,
        ],
    )
    def prop(h_hbm, src_hbm, dst_hbm, z_hbm, out_hbm, src_v, dst_v, rows_a,
             rows_b, acc_s, tab_s, gsem):
        c = lax.axis_index("c")
        s = lax.axis_index("s")
        wid = s * NC + c
        pltpu.sync_copy(src_hbm.at[wid], src_v)
        pltpu.sync_copy(dst_hbm.at[wid], dst_v)
        base = s * rows
        pltpu.sync_copy(z_hbm.at[pl.ds(base, rows)], acc_s.at[pl.ds(base, rows)])
        pltpu.sync_copy(h_hbm.at[pl.ds(base, rows)], tab_s.at[pl.ds(base, rows)])
        plsc.subcore_barrier()

        def body(i, carry):
            j0 = 2 * i
            j1 = j0 + 1
            pltpu.sync_copy(tab_s.at[src_v.at[j0]], rows_a)
            h = pltpu.async_copy(tab_s.at[src_v.at[j1]], rows_b, gsem)
            pltpu.sync_copy(rows_a, acc_s.at[dst_v.at[j0]], add=True)
            h.wait()
            pltpu.sync_copy(rows_b, acc_s.at[dst_v.at[j1]], add=True)
            return carry

        lax.fori_loop(0, nb // 2, body, 0)
        plsc.subcore_barrier()
        pltpu.sync_copy(acc_s.at[pl.ds(base, rows)], out_hbm.at[c, pl.ds(base, rows)])

    return prop


def _tc_proj_scale(hp, x, w):
    def body(hp_ref, x_ref, w_ref, dinv_ref, h1p_ref):
        deg = hp_ref[0, :, 0] + hp_ref[1, :, 0] + 1.0
        dinv = lax.rsqrt(deg)
        dinv_ref[...] = dinv[:, None]
        h = jnp.dot(x_ref[...], w_ref[...], preferred_element_type=jnp.float32)
        h1p_ref[...] = h * dinv[:, None]

    n = x.shape[0]
    return pl.pallas_call(
        body,
        out_shape=(
            jax.ShapeDtypeStruct((n, 1), jnp.float32),
            jax.ShapeDtypeStruct((n, w.shape[1]), jnp.float32),
        ),
    )(hp, x, w)


def _tc_stage2(p, hprev, dinv, b, w):
    def body(p_ref, hp_ref, dinv_ref, b_ref, w_ref, o_ref):
        dv = dinv_ref[...]
        tot = (p_ref[0] + p_ref[1] + hp_ref[...]) * dv + b_ref[...]
        hact = jnp.maximum(tot, 0.0)
        o_ref[...] = jnp.dot(hact, w_ref[...],
                             preferred_element_type=jnp.float32) * dv

    n = hprev.shape[0]
    return pl.pallas_call(
        body,
        out_shape=jax.ShapeDtypeStruct((n, w.shape[1]), jnp.float32),
    )(p, hprev, dinv, b.reshape(1, -1), w)


def _tc_stage3(p, hprev, dinv, b, w, bo):
    def body(p_ref, hp_ref, dinv_ref, b_ref, w_ref, bo_ref, o_ref):
        dv = dinv_ref[...]
        tot = (p_ref[0] + p_ref[1] + hp_ref[...]) * dv + b_ref[...]
        hact = jnp.maximum(tot, 0.0)
        o_ref[...] = jnp.dot(hact, w_ref[...],
                             preferred_element_type=jnp.float32) + bo_ref[...]

    n = hprev.shape[0]
    return pl.pallas_call(
        body,
        out_shape=jax.ShapeDtypeStruct((n, w.shape[1]), jnp.float32),
    )(p, hprev, dinv, b.reshape(1, -1), w, bo.reshape(1, -1))


def kernel(x, edge_index, W1, b1, W2, b2, W_out, b_out):
    n, d_in = x.shape
    e = edge_index.shape[1]
    n_pad = ((n + NS * 8 - 1) // (NS * 8)) * (NS * 8)
    nb = (e + NW * BSZ - 1) // (NW * BSZ)
    nb = ((nb + K - 1) // K) * K
    e_pad = NW * BSZ * nb

    xp = jnp.pad(x, ((0, n_pad - n), (0, 0)))
    src = jnp.concatenate(
        [edge_index[0], jnp.zeros((e_pad - e,), jnp.int32)]).reshape(NW, nb, BSZ)
    dst = jnp.concatenate(
        [edge_index[1], jnp.full((e_pad - e,), n, jnp.int32)]).reshape(NW, nb, BSZ)

    ones_hw = jnp.ones((BSZ, HW), jnp.float32)
    z_hw = jnp.zeros((n_pad, HW), jnp.float32)
    z1 = jnp.zeros((n_pad, W1.shape[1]), jnp.float32)
    z2 = jnp.zeros((n_pad, W2.shape[1]), jnp.float32)

    hp = _make_hist(n_pad, nb)(dst, ones_hw, z_hw)
    dinv, h1p = _tc_proj_scale(hp, xp, W1)
    p1 = _make_prop(n_pad, W1.shape[1], nb)(h1p, src, dst, z1)
    h2p = _tc_stage2(p1, h1p, dinv, b1, W2)
    p2 = _make_prop(n_pad, W2.shape[1], nb)(h2p, src, dst, z2)
    out = _tc_stage3(p2, h2p, dinv, b2, W_out, b_out)
    return out[:n]
